# Initial kernel scaffold; baseline (speedup 1.0000x reference)
import functools

import jax
import jax.numpy as jnp
import numpy as np
from jax import lax
from jax.experimental import pallas as pl
from jax.experimental.pallas import tpu as pltpu
from jax.experimental.pallas import tpu_sc as plsc

_NFS = [96, 192, 288, 384]
_NL = [10000, 1250, 156, 20]
_N_TEMB = 96
_N_EMB = 384
_BATCH = 4

_pallas_call = pl.pallas_call
_sc_kernel = pl.kernel


def _rup(x, m):
    return (x + m - 1) // m * m


@functools.lru_cache(maxsize=None)
def _make_sc_edge_agg(n_in, n_out_pad, C, CH, NCH):
    mesh = plsc.VectorSubcoreMesh(core_axis_name="c", subcore_axis_name="s")
    rpt = n_out_pad // 16

    def body(h_hbm, src_hbm, dst_hbm, zeros_hbm, out_hbm,
             srcb, dstb, rowb, acc, sem):
        cid = lax.axis_index("c")
        sid = lax.axis_index("s")
        wid = sid * 2 + cid
        pltpu.sync_copy(zeros_hbm.at[pl.ds(sid * rpt, rpt)],
                        acc.at[pl.ds(sid * rpt, rpt)])
        pltpu.sync_copy(src_hbm.at[pl.ds(wid * NCH, NCH)], srcb)
        pltpu.sync_copy(dst_hbm.at[pl.ds(wid * NCH, NCH)], dstb)
        plsc.subcore_barrier()

        def step(j, carry):
            pltpu.async_copy(h_hbm.at[srcb.at[j]], rowb, sem).wait()
            pltpu.sync_copy(rowb, acc.at[dstb.at[j]], add=True)
            return carry

        lax.fori_loop(0, NCH, step, 0)
        plsc.subcore_barrier()
        pltpu.sync_copy(acc.at[pl.ds(sid * rpt, rpt)],
                        out_hbm.at[cid].at[pl.ds(sid * rpt, rpt)])

    return _sc_kernel(
        body,
        out_type=jax.ShapeDtypeStruct((2, n_out_pad, C), jnp.float32),
        mesh=mesh,
        scratch_types=[
            pltpu.VMEM((NCH, CH), jnp.int32),
            pltpu.VMEM((NCH, CH), jnp.int32),
            pltpu.VMEM((CH, C), jnp.float32),
            pltpu.VMEM_SHARED((n_out_pad, C), jnp.float32),
            pltpu.SemaphoreType.DMA,
        ],
    )


def _sc_edge_agg(h, src2d, dst2d, n_out_pad, CH, NCH):
    zeros = jnp.zeros((n_out_pad, h.shape[1]), jnp.float32)
    fn = _make_sc_edge_agg(h.shape[0], n_out_pad, h.shape[1], CH, NCH)
    return fn(h, src2d, dst2d, zeros)


@functools.lru_cache(maxsize=None)
def _make_sc_gather(n_in, C, CH, NCH):
    mesh = plsc.VectorSubcoreMesh(core_axis_name="c", subcore_axis_name="s")
    E_pad = 32 * NCH * CH

    def body(x_hbm, idx_hbm, out_hbm, idxb, rowb, sem):
        cid = lax.axis_index("c")
        sid = lax.axis_index("s")
        wid = sid * 2 + cid
        pltpu.sync_copy(idx_hbm.at[pl.ds(wid * NCH, NCH)], idxb)

        def step(j, carry):
            pltpu.async_copy(x_hbm.at[idxb.at[j]], rowb, sem).wait()
            pltpu.sync_copy(rowb, out_hbm.at[pl.ds((wid * NCH + j) * CH, CH)])
            return carry

        lax.fori_loop(0, NCH, step, 0)

    return _sc_kernel(
        body,
        out_type=jax.ShapeDtypeStruct((E_pad, C), jnp.float32),
        mesh=mesh,
        scratch_types=[
            pltpu.VMEM((NCH, CH), jnp.int32),
            pltpu.VMEM((CH, C), jnp.float32),
            pltpu.SemaphoreType.DMA,
        ],
    )


def _sc_gather(x, idx2d, CH, NCH):
    fn = _make_sc_gather(x.shape[0], x.shape[1], CH, NCH)
    return fn(x, idx2d)


def _silu(x):
    return x * jax.nn.sigmoid(x)


def _bn_silu(x, g, be):
    m = jnp.mean(x, axis=0, keepdims=True)
    v = jnp.mean((x - m) * (x - m), axis=0, keepdims=True)
    return _silu((x - m) * jax.lax.rsqrt(v + 1e-5) * g + be)


@functools.lru_cache(maxsize=None)
def _make_tc_bn_silu(N, C, with_mm, Cout=0):
    def body_plain(x_ref, g_ref, be_ref, h_ref):
        h_ref[...] = _bn_silu(x_ref[...], g_ref[...], be_ref[...])

    def body_mm(x_ref, g_ref, be_ref, w_ref, h_ref, y_ref):
        h = _bn_silu(x_ref[...], g_ref[...], be_ref[...])
        h_ref[...] = h
        y_ref[...] = jnp.dot(h, w_ref[...], preferred_element_type=jnp.float32)

    if with_mm:
        return _pallas_call(
            body_mm,
            out_shape=(jax.ShapeDtypeStruct((N, C), jnp.float32),
                       jax.ShapeDtypeStruct((N, Cout), jnp.float32)))
    return _pallas_call(
        body_plain, out_shape=jax.ShapeDtypeStruct((N, C), jnp.float32))


def _tc_bn_silu(x, g, be):
    return _make_tc_bn_silu(x.shape[0], x.shape[1], False)(
        x, g[None, :], be[None, :])


def _tc_bn_silu_mm(x, g, be, w):
    return _make_tc_bn_silu(x.shape[0], x.shape[1], True, w.shape[1])(
        x, g[None, :], be[None, :], w)


@functools.lru_cache(maxsize=None)
def _make_tc_conv_out(N, ni, nf, n_out_pad, use_wn, extra):
    def body(*refs):
        i = 0
        h_ref = refs[i]; i += 1
        agg_ref = refs[i]; i += 1
        ws_ref = refs[i]; i += 1
        wn_ref = None
        if use_wn:
            wn_ref = refs[i]; i += 1
        b_ref = refs[i]; i += 1
        oh_ref = te_ref = sk_ref = None
        if extra == 'te':
            oh_ref = refs[i]; te_ref = refs[i + 1]; i += 2
        elif extra == 'skip':
            sk_ref = refs[i]; i += 1
        o_ref = refs[i]

        h = h_ref[...]
        a = agg_ref[0, :N, :] + agg_ref[1, :N, :]
        out = jnp.dot(h, ws_ref[...], preferred_element_type=jnp.float32)
        if use_wn:
            out = out + jnp.dot(a, wn_ref[...],
                                preferred_element_type=jnp.float32)
        else:
            out = out + a
        out = out + b_ref[...]
        if extra == 'te':
            out = out + jnp.dot(oh_ref[...], te_ref[...],
                                preferred_element_type=jnp.float32)
        elif extra == 'skip':
            out = out + sk_ref[...]
        o_ref[...] = out

    return _pallas_call(body, out_shape=jax.ShapeDtypeStruct((N, nf),
                                                             jnp.float32))


def _tc_conv_out(h, aggpair, Ws, Wn, b, oh=None, te=None, skip=None):
    N = h.shape[0]
    use_wn = Wn is not None
    extra = 'te' if te is not None else ('skip' if skip is not None else None)
    fn = _make_tc_conv_out(N, h.shape[1], Ws.shape[1], aggpair.shape[1],
                           use_wn, extra)
    args = [h, aggpair, Ws]
    if use_wn:
        args.append(Wn)
    args.append(b[None, :])
    if extra == 'te':
        args += [oh, te]
    elif extra == 'skip':
        args.append(skip)
    return fn(*args)


@functools.lru_cache(maxsize=None)
def _make_tc_matmul(N, ni, nf):
    def body(x_ref, w_ref, b_ref, o_ref):
        o_ref[...] = jnp.dot(x_ref[...], w_ref[...],
                             preferred_element_type=jnp.float32) + b_ref[...]

    return _pallas_call(body, out_shape=jax.ShapeDtypeStruct((N, nf),
                                                             jnp.float32))


def _tc_matmul(x, w, b):
    return _make_tc_matmul(x.shape[0], x.shape[1], w.shape[1])(x, w, b[None, :])


def _onehot_from_pres(pres):
    b3 = pres[:, 3:4] > 0
    b2 = (pres[:, 2:3] > 0) & ~b3
    b1 = (pres[:, 1:2] > 0) & ~b3 & ~b2
    b0 = ~(b3 | b2 | b1)
    return jnp.concatenate([b0, b1, b2, b3], axis=1).astype(jnp.float32)


@functools.lru_cache(maxsize=None)
def _make_tc_pool_fin(N2, n2_pad, C, nf):
    def body(agg_ref, w_ref, b_ref, y_ref, oh_ref):
        s = agg_ref[0, :N2, :] + agg_ref[1, :N2, :]
        xs = s[:, :C]
        pres = s[:, C:C + 4]
        cnt = jnp.maximum(s[:, C + 4:C + 5], 1.0)
        xm = xs / cnt
        y_ref[...] = jnp.dot(xm, w_ref[...],
                             preferred_element_type=jnp.float32) + b_ref[...]
        oh_ref[...] = _onehot_from_pres(pres)

    return _pallas_call(
        body,
        out_shape=(jax.ShapeDtypeStruct((N2, nf), jnp.float32),
                   jax.ShapeDtypeStruct((N2, 4), jnp.float32)))


@functools.lru_cache(maxsize=None)
def _make_tc_emb(sum_nf):
    def body(t_ref, w1_ref, b1_ref, w2_ref, b2_ref, wt_ref, bt_ref, te_ref):
        half = _N_TEMB // 2
        freqs = jnp.exp(
            -np.log(10000.0)
            * jax.lax.broadcasted_iota(jnp.float32, (1, half), 1) / half)
        args = t_ref[...] * freqs
        temb = jnp.concatenate([jnp.cos(args), jnp.sin(args)], axis=1)
        h = _silu(jnp.dot(temb, w1_ref[...],
                          preferred_element_type=jnp.float32) + b1_ref[...])
        emb = _silu(jnp.dot(h, w2_ref[...],
                            preferred_element_type=jnp.float32) + b2_ref[...])
        se = _silu(emb)
        te_ref[...] = jnp.dot(se, wt_ref[...],
                              preferred_element_type=jnp.float32) + bt_ref[...]

    return _pallas_call(body, out_shape=jax.ShapeDtypeStruct((8, sum_nf),
                                                             jnp.float32))


@functools.lru_cache(maxsize=None)
def _make_tc_final(N, C):
    def body(x_ref, g_ref, be_ref, w_ref, o_ref):
        h = _bn_silu(x_ref[...], g_ref[...], be_ref[...])
        o_ref[...] = jnp.dot(h, w_ref[...], preferred_element_type=jnp.float32)

    return _pallas_call(body, out_shape=jax.ShapeDtypeStruct((N, 3),
                                                             jnp.float32))


_AGG_CFG = [(128, 40), (128, 5), (80, 1), (16, 1)]
_CL_CFG = [(64, 5), (40, 1), (16, 1)]


def _prep_pair(src, dst, n_out, CH, NCH):
    E = src.shape[0]
    E_pad = 32 * CH * NCH
    src = jnp.concatenate(
        [src.astype(jnp.int32), jnp.zeros((E_pad - E,), jnp.int32)])
    dst = jnp.concatenate(
        [dst.astype(jnp.int32), jnp.full((E_pad - E,), n_out, jnp.int32)])
    return src.reshape(32 * NCH, CH), dst.reshape(32 * NCH, CH)


def _prep_idx(idx, CH, NCH):
    E = idx.shape[0]
    E_pad = 32 * CH * NCH
    idx = jnp.concatenate(
        [idx.astype(jnp.int32), jnp.zeros((E_pad - E,), jnp.int32)])
    return idx.reshape(32 * NCH, CH)


def _agg_level(h, lev, eprep, n_out_pad):
    CH, NCH = _AGG_CFG[lev]
    return _sc_edge_agg(h, eprep[0], eprep[1], n_out_pad, CH, NCH)


def _resblock(x_in, lev, eprep, n_pad, oh, te, p):
    ni = x_in.shape[1]
    nf = p['c1']['Ws'].shape[1]
    if ni <= nf:
        h1 = _tc_bn_silu(x_in, p['c1']['g'], p['c1']['be'])
        agg = _agg_level(h1, lev, eprep, n_pad)
        x = _tc_conv_out(h1, agg, p['c1']['Ws'], p['c1']['Wn'], p['c1']['b'],
                         oh=oh, te=te)
    else:
        h1, y1 = _tc_bn_silu_mm(x_in, p['c1']['g'], p['c1']['be'],
                                p['c1']['Wn'])
        agg = _agg_level(y1, lev, eprep, n_pad)
        x = _tc_conv_out(h1, agg, p['c1']['Ws'], None, p['c1']['b'],
                         oh=oh, te=te)
    h2 = _tc_bn_silu(x, p['c2']['g'], p['c2']['be'])
    agg2 = _agg_level(h2, lev, eprep, n_pad)
    if 'id' in p:
        skip = _tc_matmul(x_in, p['id']['W'], p['id']['b'])
    else:
        skip = x_in
    return _tc_conv_out(h2, agg2, p['c2']['Ws'], p['c2']['Wn'], p['c2']['b'],
                        skip=skip)


def kernel(feats, t, batch_idx, edge_index0, edge_index1, edge_index2,
           edge_index3, cluster0, cluster1, cluster2, params):
    eis = [edge_index0, edge_index1, edge_index2, edge_index3]
    clusters = [cluster0, cluster1, cluster2]

    n_pads = [_rup(n + 1, 16) for n in _NL]

    epreps = []
    for lev in range(4):
        CH, NCH = _AGG_CFG[lev]
        epreps.append(_prep_pair(eis[lev][0], eis[lev][1], _NL[lev], CH, NCH))
    pool_preps = []
    un_preps = []
    for lev in range(3):
        CH, NCH = _CL_CFG[lev]
        src = jnp.arange(_NL[lev], dtype=jnp.int32)
        pool_preps.append(_prep_pair(src, clusters[lev], _NL[lev + 1], CH, NCH))
        un_preps.append(_prep_idx(clusters[lev], CH, NCH))

    res_ps = ([params['downs'][i]['res'][0] for i in range(4)]
              + [params['mid']]
              + [params['ups'][i]['res'][j] for i in range(4) for j in range(2)])
    nfs = [p['c1']['Ws'].shape[1] for p in res_ps]
    wt_cat = jnp.concatenate([p['t']['W'] for p in res_ps], axis=1)
    bt_cat = jnp.concatenate([p['t']['b'] for p in res_ps], axis=0)
    offs = np.concatenate([[0], np.cumsum(nfs)]).astype(int)
    t_pad = jnp.zeros((8, 1), jnp.float32).at[:_BATCH, 0].set(
        t.astype(jnp.float32))
    te_all = _make_tc_emb(int(offs[-1]))(
        t_pad, params['emb1']['W'], params['emb1']['b'][None, :],
        params['emb2']['W'], params['emb2']['b'][None, :], wt_cat,
        bt_cat[None, :])
    tes = [te_all[:_BATCH, offs[k]:offs[k + 1]] for k in range(len(res_ps))]

    oh = [jax.nn.one_hot(batch_idx, 4, dtype=jnp.float32)]

    feats_p = jnp.pad(feats, ((0, 0), (0, 13)))
    ws_in = jnp.pad(params['conv_in']['Ws'], ((0, 13), (0, 0)))
    wn_in = jnp.pad(params['conv_in']['Wn'], ((0, 13), (0, 0)))
    agg_in = _agg_level(feats_p, 0, epreps[0], n_pads[0])
    x = _tc_conv_out(feats_p, agg_in, ws_in, wn_in, params['conv_in']['b'])

    saved = [x]
    for i in range(4):
        x = _resblock(x, i, epreps[i], n_pads[i], oh[i], tes[i],
                      params['downs'][i]['res'][0])
        saved.append(x)
        if i != 3:
            C = x.shape[1]
            xa = jnp.concatenate(
                [x, oh[i], jnp.ones((_NL[i], 1), jnp.float32),
                 jnp.zeros((_NL[i], 11), jnp.float32)], axis=1)
            CH, NCH = _CL_CFG[i]
            aggp = _sc_edge_agg(xa, pool_preps[i][0], pool_preps[i][1],
                                n_pads[i + 1], CH, NCH)
            pd = params['downs'][i]['down']
            fin = _make_tc_pool_fin(_NL[i + 1], n_pads[i + 1], C,
                                    pd['W'].shape[1])
            x, oh_next = fin(aggp, pd['W'], pd['b'][None, :])
            oh.append(oh_next)
            saved.append(x)

    x = _resblock(x, 3, epreps[3], n_pads[3], oh[3], tes[4], params['mid'])

    for i in range(4):
        lev = 3 - i
        for j in range(2):
            skip = saved.pop()
            p = params['ups'][i]['res'][j]
            x = _resblock(jnp.concatenate([x, skip], axis=-1), lev,
                          epreps[lev], n_pads[lev], oh[lev],
                          tes[5 + 2 * i + j], p)
        if i != 3:
            pu = params['ups'][i]['up']
            y = _tc_matmul(x, pu['W'], pu['b'])
            CH, NCH = _CL_CFG[lev - 1]
            g = _sc_gather(y, un_preps[lev - 1], CH, NCH)
            x = g[:_NL[lev - 1]]

    po = params['out']
    return _make_tc_final(_NL[0], _NFS[0])(
        x, po['g'][None, :], po['be'][None, :], po['W'])


# speedup vs baseline: 2.6987x; 2.6987x over previous
import functools

import jax
import jax.numpy as jnp
import numpy as np
from jax import lax
from jax.experimental import pallas as pl
from jax.experimental.pallas import tpu as pltpu
from jax.experimental.pallas import tpu_sc as plsc

_NFS = [96, 192, 288, 384]
_NL = [10000, 1250, 156, 20]
_N_TEMB = 96
_N_EMB = 384
_BATCH = 4

_pallas_call = pl.pallas_call
_sc_kernel = pl.kernel


def _rup(x, m):
    return (x + m - 1) // m * m


@functools.lru_cache(maxsize=None)
def _make_sc_edge_agg(n_in, n_out_pad, C, CH, NCH):
    mesh = plsc.VectorSubcoreMesh(core_axis_name="c", subcore_axis_name="s")
    rpt = n_out_pad // 16

    def body(h_hbm, src_hbm, dst_hbm, zeros_hbm, out_hbm,
             srcb, dstb, rowb, acc, sem):
        cid = lax.axis_index("c")
        sid = lax.axis_index("s")
        wid = sid * 2 + cid
        pltpu.sync_copy(zeros_hbm.at[pl.ds(sid * rpt, rpt)],
                        acc.at[pl.ds(sid * rpt, rpt)])
        for j in range(NCH):
            pltpu.sync_copy(src_hbm.at[pl.ds((wid * NCH + j) * CH, CH)],
                            srcb.at[j])
            pltpu.sync_copy(dst_hbm.at[pl.ds((wid * NCH + j) * CH, CH)],
                            dstb.at[j])
        plsc.subcore_barrier()

        def step(j, carry):
            pltpu.async_copy(h_hbm.at[srcb.at[j]], rowb, sem).wait()
            pltpu.sync_copy(rowb, acc.at[dstb.at[j]], add=True)
            return carry

        lax.fori_loop(0, NCH, step, 0)
        plsc.subcore_barrier()
        pltpu.sync_copy(acc.at[pl.ds(sid * rpt, rpt)],
                        out_hbm.at[cid].at[pl.ds(sid * rpt, rpt)])

    return _sc_kernel(
        body,
        out_type=jax.ShapeDtypeStruct((2, n_out_pad, C), jnp.float32),
        mesh=mesh,
        compiler_params=pltpu.CompilerParams(use_tc_tiling_on_sc=False),
        scratch_types=[
            pltpu.VMEM((NCH, CH), jnp.int32),
            pltpu.VMEM((NCH, CH), jnp.int32),
            pltpu.VMEM((CH, C), jnp.float32),
            pltpu.VMEM_SHARED((n_out_pad, C), jnp.float32),
            pltpu.SemaphoreType.DMA,
        ],
    )


def _sc_edge_agg(h, src2d, dst2d, n_out_pad, CH, NCH):
    zeros = jnp.zeros((n_out_pad, h.shape[1]), jnp.float32)
    fn = _make_sc_edge_agg(h.shape[0], n_out_pad, h.shape[1], CH, NCH)
    return fn(h, src2d, dst2d, zeros)


@functools.lru_cache(maxsize=None)
def _make_sc_gather(n_in, C, CH, NCH):
    mesh = plsc.VectorSubcoreMesh(core_axis_name="c", subcore_axis_name="s")
    E_pad = 32 * NCH * CH

    def body(x_hbm, idx_hbm, out_hbm, idxb, rowb, sem):
        cid = lax.axis_index("c")
        sid = lax.axis_index("s")
        wid = sid * 2 + cid
        for j in range(NCH):
            pltpu.sync_copy(idx_hbm.at[pl.ds((wid * NCH + j) * CH, CH)],
                            idxb.at[j])

        def step(j, carry):
            pltpu.async_copy(x_hbm.at[idxb.at[j]], rowb, sem).wait()
            pltpu.sync_copy(rowb, out_hbm.at[pl.ds((wid * NCH + j) * CH, CH)])
            return carry

        lax.fori_loop(0, NCH, step, 0)

    return _sc_kernel(
        body,
        out_type=jax.ShapeDtypeStruct((E_pad, C), jnp.float32),
        mesh=mesh,
        compiler_params=pltpu.CompilerParams(use_tc_tiling_on_sc=False),
        scratch_types=[
            pltpu.VMEM((NCH, CH), jnp.int32),
            pltpu.VMEM((CH, C), jnp.float32),
            pltpu.SemaphoreType.DMA,
        ],
    )


def _sc_gather(x, idx2d, CH, NCH):
    fn = _make_sc_gather(x.shape[0], x.shape[1], CH, NCH)
    return fn(x, idx2d)


def _dot(a, b, preferred_element_type=jnp.float32):
    return jax.lax.dot_general(
        a, b, (((1,), (0,)), ((), ())),
        precision=jax.lax.Precision.DEFAULT,
        preferred_element_type=preferred_element_type)


def _dot_hi(a, b):
    return jax.lax.dot_general(
        a, b, (((1,), (0,)), ((), ())),
        precision=jax.lax.Precision.HIGHEST,
        preferred_element_type=jnp.float32)

def _silu(x):
    return x * jax.nn.sigmoid(x)


def _bn_silu(x, g, be):
    m = jnp.mean(x, axis=0, keepdims=True)
    v = jnp.mean((x - m) * (x - m), axis=0, keepdims=True)
    return _silu((x - m) * jax.lax.rsqrt(v + 1e-5) * g + be)


@functools.lru_cache(maxsize=None)
def _make_tc_bn_silu(N, C, with_mm, Cout=0):
    def body_plain(x_ref, g_ref, be_ref, h_ref):
        h_ref[...] = _bn_silu(x_ref[...], g_ref[...], be_ref[...])

    def body_mm(x_ref, g_ref, be_ref, w_ref, h_ref, y_ref):
        h = _bn_silu(x_ref[...], g_ref[...], be_ref[...])
        h_ref[...] = h
        y_ref[...] = _dot(h, w_ref[...], preferred_element_type=jnp.float32)

    if with_mm:
        return _pallas_call(
            body_mm,
            out_shape=(jax.ShapeDtypeStruct((N, C), jnp.float32),
                       jax.ShapeDtypeStruct((N, Cout), jnp.float32)))
    return _pallas_call(
        body_plain, out_shape=jax.ShapeDtypeStruct((N, C), jnp.float32))


_BLK = 2000


@functools.lru_cache(maxsize=None)
def _make_tc_colstats(N, C):
    def body(x_ref, o_ref):
        i = pl.program_id(0)

        @pl.when(i == 0)
        def _():
            o_ref[...] = jnp.zeros_like(o_ref)

        x = x_ref[...]
        o_ref[0:1, :] += jnp.sum(x, axis=0, keepdims=True)
        o_ref[1:2, :] += jnp.sum(x * x, axis=0, keepdims=True)

    return _pallas_call(
        body,
        grid=(N // _BLK,),
        in_specs=[pl.BlockSpec((_BLK, C), lambda i: (i, 0))],
        out_specs=pl.BlockSpec((8, C), lambda i: (0, 0)),
        out_shape=jax.ShapeDtypeStruct((8, C), jnp.float32))


@functools.lru_cache(maxsize=None)
def _make_tc_bn_apply(N, C, with_mm, Cout=0):
    def norm(x, st_ref, g_ref, be_ref):
        m = st_ref[0:1, :] / N
        v = st_ref[1:2, :] / N - m * m
        return _silu((x - m) * jax.lax.rsqrt(v + 1e-5) * g_ref[...]
                     + be_ref[...])

    def body_plain(x_ref, st_ref, g_ref, be_ref, h_ref):
        h_ref[...] = norm(x_ref[...], st_ref, g_ref, be_ref)

    def body_mm(x_ref, st_ref, g_ref, be_ref, w_ref, h_ref, y_ref):
        h = norm(x_ref[...], st_ref, g_ref, be_ref)
        h_ref[...] = h
        y_ref[...] = _dot(h, w_ref[...], preferred_element_type=jnp.float32)

    in_specs = [
        pl.BlockSpec((_BLK, C), lambda i: (i, 0)),
        pl.BlockSpec((8, C), lambda i: (0, 0)),
        pl.BlockSpec((1, C), lambda i: (0, 0)),
        pl.BlockSpec((1, C), lambda i: (0, 0)),
    ]
    if with_mm:
        return _pallas_call(
            body_mm,
            grid=(N // _BLK,),
            in_specs=in_specs + [pl.BlockSpec((C, Cout), lambda i: (0, 0))],
            out_specs=(pl.BlockSpec((_BLK, C), lambda i: (i, 0)),
                       pl.BlockSpec((_BLK, Cout), lambda i: (i, 0))),
            out_shape=(jax.ShapeDtypeStruct((N, C), jnp.float32),
                       jax.ShapeDtypeStruct((N, Cout), jnp.float32)))
    return _pallas_call(
        body_plain,
        grid=(N // _BLK,),
        in_specs=in_specs,
        out_specs=pl.BlockSpec((_BLK, C), lambda i: (i, 0)),
        out_shape=jax.ShapeDtypeStruct((N, C), jnp.float32))


def _tc_bn_silu(x, g, be):
    N, C = x.shape
    if N >= 8000:
        st = _make_tc_colstats(N, C)(x)
        return _make_tc_bn_apply(N, C, False)(x, st, g[None, :], be[None, :])
    return _make_tc_bn_silu(N, C, False)(x, g[None, :], be[None, :])


def _tc_bn_silu_mm(x, g, be, w):
    N, C = x.shape
    if N >= 8000:
        st = _make_tc_colstats(N, C)(x)
        return _make_tc_bn_apply(N, C, True, w.shape[1])(
            x, st, g[None, :], be[None, :], w)
    return _make_tc_bn_silu(N, C, True, w.shape[1])(
        x, g[None, :], be[None, :], w)


@functools.lru_cache(maxsize=None)
def _make_tc_conv_out(N, ni, nf, n_out_pad, use_wn, extra, hi=False):
    def body(*refs):
        i = 0
        h_ref = refs[i]; i += 1
        agg_ref = refs[i]; i += 1
        ws_ref = refs[i]; i += 1
        wn_ref = None
        if use_wn:
            wn_ref = refs[i]; i += 1
        b_ref = refs[i]; i += 1
        oh_ref = te_ref = sk_ref = None
        if extra == 'te':
            oh_ref = refs[i]; te_ref = refs[i + 1]; i += 2
        elif extra == 'skip':
            sk_ref = refs[i]; i += 1
        o_ref = refs[i]

        h = h_ref[...]
        if agg_ref.shape[1] == n_out_pad:
            a = agg_ref[0, :N, :] + agg_ref[1, :N, :]
        else:
            a = agg_ref[0] + agg_ref[1]
        dot0 = _dot_hi if hi else _dot
        out = dot0(h, ws_ref[...])
        if use_wn:
            out = out + _dot(a, wn_ref[...],
                                preferred_element_type=jnp.float32)
        else:
            out = out + a
        out = out + b_ref[...]
        if extra == 'te':
            out = out + _dot_hi(oh_ref[...], te_ref[...])
        elif extra == 'skip':
            out = out + sk_ref[...]
        o_ref[...] = out

    if N >= 8000:
        BLK = 2000
        c_agg = ni if use_wn else nf
        in_specs = [
            pl.BlockSpec((BLK, ni), lambda i: (i, 0)),
            pl.BlockSpec((2, BLK, c_agg), lambda i: (0, i, 0)),
            pl.BlockSpec((ni, nf), lambda i: (0, 0)),
        ]
        if use_wn:
            in_specs.append(pl.BlockSpec((ni, nf), lambda i: (0, 0)))
        in_specs.append(pl.BlockSpec((1, nf), lambda i: (0, 0)))
        if extra == 'te':
            in_specs.append(pl.BlockSpec((BLK, 4), lambda i: (i, 0)))
            in_specs.append(pl.BlockSpec((4, nf), lambda i: (0, 0)))
        elif extra == 'skip':
            in_specs.append(pl.BlockSpec((BLK, nf), lambda i: (i, 0)))
        return _pallas_call(
            body,
            grid=(N // BLK,),
            in_specs=in_specs,
            out_specs=pl.BlockSpec((BLK, nf), lambda i: (i, 0)),
            out_shape=jax.ShapeDtypeStruct((N, nf), jnp.float32))
    return _pallas_call(body, out_shape=jax.ShapeDtypeStruct((N, nf),
                                                             jnp.float32))


def _tc_conv_out(h, aggpair, Ws, Wn, b, oh=None, te=None, skip=None,
                 hi=False):
    N = h.shape[0]
    use_wn = Wn is not None
    extra = 'te' if te is not None else ('skip' if skip is not None else None)
    fn = _make_tc_conv_out(N, h.shape[1], Ws.shape[1], aggpair.shape[1],
                           use_wn, extra, hi)
    args = [h, aggpair, Ws]
    if use_wn:
        args.append(Wn)
    args.append(b[None, :])
    if extra == 'te':
        args += [oh, te]
    elif extra == 'skip':
        args.append(skip)
    return fn(*args)


@functools.lru_cache(maxsize=None)
def _make_tc_matmul(N, ni, nf, hi=False):
    dot = _dot_hi if hi else _dot

    def body(x_ref, w_ref, b_ref, o_ref):
        o_ref[...] = dot(x_ref[...], w_ref[...]) + b_ref[...]

    if N >= 8000:
        return _pallas_call(
            body,
            grid=(N // _BLK,),
            in_specs=[pl.BlockSpec((_BLK, ni), lambda i: (i, 0)),
                      pl.BlockSpec((ni, nf), lambda i: (0, 0)),
                      pl.BlockSpec((1, nf), lambda i: (0, 0))],
            out_specs=pl.BlockSpec((_BLK, nf), lambda i: (i, 0)),
            out_shape=jax.ShapeDtypeStruct((N, nf), jnp.float32))
    return _pallas_call(body, out_shape=jax.ShapeDtypeStruct((N, nf),
                                                             jnp.float32))


def _tc_matmul(x, w, b, hi=False):
    return _make_tc_matmul(x.shape[0], x.shape[1], w.shape[1], hi)(
        x, w, b[None, :])


def _onehot_from_pres(pres):
    f3 = (pres[:, 3:4] > 0).astype(jnp.float32)
    f2 = (pres[:, 2:3] > 0).astype(jnp.float32) * (1.0 - f3)
    f1 = (pres[:, 1:2] > 0).astype(jnp.float32) * (1.0 - f3 - f2)
    f0 = 1.0 - f3 - f2 - f1
    return jnp.concatenate([f0, f1, f2, f3], axis=1)


@functools.lru_cache(maxsize=None)
def _make_tc_pool_fin(N2, n2_pad, C, nf):
    def body(agg_ref, w_ref, b_ref, y_ref, oh_ref):
        s = agg_ref[0, :N2, :] + agg_ref[1, :N2, :]
        xs = s[:, :C]
        pres = s[:, C:C + 4]
        cnt = jnp.maximum(s[:, C + 4:C + 5], 1.0)
        xm = xs / cnt
        y_ref[...] = _dot(xm, w_ref[...],
                             preferred_element_type=jnp.float32) + b_ref[...]
        oh_ref[...] = _onehot_from_pres(pres)

    return _pallas_call(
        body,
        out_shape=(jax.ShapeDtypeStruct((N2, nf), jnp.float32),
                   jax.ShapeDtypeStruct((N2, 4), jnp.float32)))


@functools.lru_cache(maxsize=None)
def _make_tc_emb(sum_nf):
    def body(t_ref, w1_ref, b1_ref, w2_ref, b2_ref, wt_ref, bt_ref, te_ref):
        half = _N_TEMB // 2
        freqs = jnp.exp(
            -np.log(10000.0)
            * jax.lax.broadcasted_iota(jnp.int32, (1, half), 1)
            .astype(jnp.float32) / half)
        args = t_ref[...] * freqs
        temb = jnp.concatenate([jnp.cos(args), jnp.sin(args)], axis=1)
        h = _silu(_dot(temb, w1_ref[...],
                          preferred_element_type=jnp.float32) + b1_ref[...])
        emb = _silu(_dot(h, w2_ref[...],
                            preferred_element_type=jnp.float32) + b2_ref[...])
        se = _silu(emb)
        te_ref[...] = _dot(se, wt_ref[...],
                              preferred_element_type=jnp.float32) + bt_ref[...]

    return _pallas_call(body, out_shape=jax.ShapeDtypeStruct((8, sum_nf),
                                                             jnp.float32))


@functools.lru_cache(maxsize=None)
def _make_tc_final(N, C):
    def body(x_ref, g_ref, be_ref, w_ref, o_ref):
        h = _bn_silu(x_ref[...], g_ref[...], be_ref[...])
        o_ref[...] = _dot(h, w_ref[...], preferred_element_type=jnp.float32)

    return _pallas_call(body, out_shape=jax.ShapeDtypeStruct((N, 3),
                                                             jnp.float32))


_AGG_CFG = [(128, 40), (128, 5), (80, 1), (16, 1)]
_CL_CFG = [(64, 5), (40, 1), (16, 1)]


def _prep_pair(src, dst, n_out, CH, NCH):
    E = src.shape[0]
    E_pad = 32 * CH * NCH
    src = jnp.concatenate(
        [src.astype(jnp.int32), jnp.zeros((E_pad - E,), jnp.int32)])
    dst = jnp.concatenate(
        [dst.astype(jnp.int32), jnp.full((E_pad - E,), n_out, jnp.int32)])
    return src, dst


def _prep_idx(idx, CH, NCH):
    E = idx.shape[0]
    E_pad = 32 * CH * NCH
    return jnp.concatenate(
        [idx.astype(jnp.int32), jnp.zeros((E_pad - E,), jnp.int32)])


def _agg_level(h, lev, eprep, n_out_pad):
    CH, NCH = _AGG_CFG[lev]
    return _sc_edge_agg(h, eprep[0], eprep[1], n_out_pad, CH, NCH)


def _resblock(x_in, lev, eprep, n_pad, oh, te, p):
    h1, y1 = _tc_bn_silu_mm(x_in, p['c1']['g'], p['c1']['be'], p['c1']['Wn'])
    agg = _agg_level(y1, lev, eprep, n_pad)
    x = _tc_conv_out(h1, agg, p['c1']['Ws'], None, p['c1']['b'],
                     oh=oh, te=te)
    h2, y2 = _tc_bn_silu_mm(x, p['c2']['g'], p['c2']['be'], p['c2']['Wn'])
    agg2 = _agg_level(y2, lev, eprep, n_pad)
    if 'id' in p:
        skip = _tc_matmul(x_in, p['id']['W'], p['id']['b'])
    else:
        skip = x_in
    return _tc_conv_out(h2, agg2, p['c2']['Ws'], None, p['c2']['b'],
                        skip=skip)


def kernel(feats, t, batch_idx, edge_index0, edge_index1, edge_index2,
           edge_index3, cluster0, cluster1, cluster2, params):
    eis = [edge_index0, edge_index1, edge_index2, edge_index3]
    clusters = [cluster0, cluster1, cluster2]

    n_pads = [_rup(n + 1, 128) for n in _NL]

    epreps = []
    for lev in range(4):
        CH, NCH = _AGG_CFG[lev]
        epreps.append(_prep_pair(eis[lev][0], eis[lev][1], _NL[lev], CH, NCH))
    pool_preps = []
    un_preps = []
    for lev in range(3):
        CH, NCH = _CL_CFG[lev]
        src = jnp.arange(_NL[lev], dtype=jnp.int32)
        pool_preps.append(_prep_pair(src, clusters[lev], _NL[lev + 1], CH, NCH))
        un_preps.append(_prep_idx(clusters[lev], CH, NCH))

    res_ps = ([params['downs'][i]['res'][0] for i in range(4)]
              + [params['mid']]
              + [params['ups'][i]['res'][j] for i in range(4) for j in range(2)])
    nfs = [p['c1']['Ws'].shape[1] for p in res_ps]
    wt_cat = jnp.concatenate([p['t']['W'] for p in res_ps], axis=1)
    bt_cat = jnp.concatenate([p['t']['b'] for p in res_ps], axis=0)
    offs = np.concatenate([[0], np.cumsum(nfs)]).astype(int)
    t_pad = jnp.zeros((8, 1), jnp.float32).at[:_BATCH, 0].set(
        t.astype(jnp.float32))
    te_all = _make_tc_emb(int(offs[-1]))(
        t_pad, params['emb1']['W'], params['emb1']['b'][None, :],
        params['emb2']['W'], params['emb2']['b'][None, :], wt_cat,
        bt_cat[None, :])
    tes = [te_all[:_BATCH, offs[k]:offs[k + 1]] for k in range(len(res_ps))]

    oh = [jax.nn.one_hot(batch_idx, 4, dtype=jnp.float32)]

    feats_p = jnp.pad(feats, ((0, 0), (0, 13)))
    ws_in = jnp.pad(params['conv_in']['Ws'], ((0, 13), (0, 0)))
    wn_in = jnp.pad(params['conv_in']['Wn'], ((0, 13), (0, 0)))
    y_in = _tc_matmul(feats_p, wn_in, jnp.zeros((_NFS[0],), jnp.float32), hi=True)
    agg_in = _agg_level(y_in, 0, epreps[0], n_pads[0])
    x = _tc_conv_out(feats_p, agg_in, ws_in, None, params['conv_in']['b'], hi=True)

    saved = [x]
    for i in range(4):
        x = _resblock(x, i, epreps[i], n_pads[i], oh[i], tes[i],
                      params['downs'][i]['res'][0])
        saved.append(x)
        if i != 3:
            C = x.shape[1]
            xa = jnp.concatenate(
                [x, oh[i], jnp.ones((_NL[i], 1), jnp.float32),
                 jnp.zeros((_NL[i], 11), jnp.float32)], axis=1)
            CH, NCH = _CL_CFG[i]
            aggp = _sc_edge_agg(xa, pool_preps[i][0], pool_preps[i][1],
                                n_pads[i + 1], CH, NCH)
            pd = params['downs'][i]['down']
            fin = _make_tc_pool_fin(_NL[i + 1], n_pads[i + 1], C,
                                    pd['W'].shape[1])
            x, oh_next = fin(aggp, pd['W'], pd['b'][None, :])
            oh.append(oh_next)
            saved.append(x)

    x = _resblock(x, 3, epreps[3], n_pads[3], oh[3], tes[4], params['mid'])

    for i in range(4):
        lev = 3 - i
        for j in range(2):
            skip = saved.pop()
            p = params['ups'][i]['res'][j]
            x = _resblock(jnp.concatenate([x, skip], axis=-1), lev,
                          epreps[lev], n_pads[lev], oh[lev],
                          tes[5 + 2 * i + j], p)
        if i != 3:
            pu = params['ups'][i]['up']
            y = _tc_matmul(x, pu['W'], pu['b'])
            CH, NCH = _CL_CFG[lev - 1]
            g = _sc_gather(y, un_preps[lev - 1], CH, NCH)
            x = g[:_NL[lev - 1]]

    po = params['out']
    return _make_tc_final(_NL[0], _NFS[0])(
        x, po['g'][None, :], po['be'][None, :], po['W'])
